# baseline (speedup 1.0000x reference)
import functools

import jax
import jax.numpy as jnp
from jax import lax
from jax.experimental import pallas as pl
from jax.experimental.pallas import tpu as pltpu
from jax.experimental.pallas import tpu_sc as plsc

EPS_ = 1.0

_NW = 32
_NPIX = 4096
_CHUNKS = 2


def _dice_tc_kernel(x_ref, t_ref, out_ref, acc_ref):
    x = x_ref[0]
    t = t_ref[0]
    C, HT, W = x.shape

    ipart = jnp.where(t == 0, x[0], 0.0)
    for c in range(1, C):
        ipart = ipart + jnp.where(t == c, x[c], 0.0)
    i_vec = jnp.sum(ipart, axis=0, keepdims=True)

    x2 = x.reshape(C * HT, W)
    ones = jnp.ones((1, C * HT), dtype=jnp.float32)
    s_vec = jax.lax.dot_general(
        ones, x2, (((1,), (0,)), ((), ())),
        preferred_element_type=jnp.float32)

    nv_vec = jnp.sum(jnp.where((t >= 0) & (t < C), 1.0, 0.0),
                     axis=0, keepdims=True)

    upd = jnp.concatenate([s_vec, i_vec, nv_vec], axis=0)

    @pl.when(pl.program_id(0) == 0)
    def _init():
        acc_ref[...] = upd

    @pl.when(pl.program_id(0) != 0)
    def _acc():
        acc_ref[...] += upd

    @pl.when(pl.program_id(0) == pl.num_programs(0) - 1)
    def _fin():
        acc = acc_ref[...]
        out_ref[0, 0] = jnp.sum(acc[0])
        out_ref[0, 1] = jnp.sum(acc[1])
        out_ref[0, 2] = jnp.sum(acc[2])


def _make_sc_part(C, H, W, b_sc):
    mesh = plsc.VectorSubcoreMesh(core_axis_name="c", subcore_axis_name="s",
                                  num_cores=2, num_subcores=16)
    ROWS = _NPIX // W

    @functools.partial(
        pl.kernel,
        out_type=jax.ShapeDtypeStruct((_NW * 48,), jnp.float32),
        mesh=mesh,
        scratch_types=[
            pltpu.VMEM((C * ROWS, W), jnp.float32),
            pltpu.VMEM((ROWS, W), jnp.int32),
            pltpu.VMEM((48,), jnp.float32),
            pltpu.SemaphoreType.DMA,
        ],
    )
    def _sc_part(x_hbm, t_hbm, out_hbm, xbuf, tbuf, part, sem):
        wid = lax.axis_index("s") * 2 + lax.axis_index("c")
        zero = jnp.zeros((16,), jnp.float32)

        def chunk(k, carry):
            r0 = (wid * _CHUNKS + k) * ROWS
            copies = [
                pltpu.async_copy(
                    x_hbm.at[pl.ds((b_sc * C + c) * H + r0, ROWS), :],
                    xbuf.at[pl.ds(c * ROWS, ROWS), :], sem)
                for c in range(C)
            ]
            tcp = pltpu.async_copy(t_hbm.at[pl.ds(b_sc * H + r0, ROWS), :],
                                   tbuf, sem)
            for cp in copies:
                cp.wait()
            tcp.wait()

            def row_body(r, carry_r):
                def col_body(j, carry2):
                    ssum, isum, nvsum = carry2
                    tv = tbuf[r, pl.ds(j * 16, 16)]
                    valid = (tv >= 0) & (tv < C)
                    nvsum = nvsum + jnp.where(valid, 1.0, 0.0)
                    for c in range(C):
                        xv = xbuf[c * ROWS + r, pl.ds(j * 16, 16)]
                        ssum = ssum + xv
                        isum = isum + jnp.where(tv == c, xv, 0.0)
                    return (ssum, isum, nvsum)

                return lax.fori_loop(0, W // 16, col_body, carry_r)

            return lax.fori_loop(0, ROWS, row_body, carry)

        ssum, isum, nvsum = lax.fori_loop(0, _CHUNKS, chunk, (zero, zero, zero))
        part[pl.ds(0, 16)] = ssum
        part[pl.ds(16, 16)] = isum
        part[pl.ds(32, 16)] = nvsum
        pltpu.sync_copy(part, out_hbm.at[pl.ds(wid * 48, 48)])

    return _sc_part


def kernel(y_pred, y_true):
    B, C, H, W = y_pred.shape
    HW = H * W
    HT = 256
    GH = H // HT
    n_tc = (B - 1) * GH

    xr = y_pred.reshape(B * C * H, W)
    tr = y_true.astype(jnp.int32).reshape(B * H, W)
    sc_parts = _make_sc_part(C, H, W, B - 1)(xr, tr)

    tc_out = pl.pallas_call(
        _dice_tc_kernel,
        grid=(n_tc,),
        in_specs=[
            pl.BlockSpec((1, C, HT, W), lambda i: (i // GH, 0, i % GH, 0)),
            pl.BlockSpec((1, HT, W), lambda i: (i // GH, i % GH, 0)),
        ],
        out_specs=pl.BlockSpec((1, 3), lambda i: (0, 0), memory_space=pltpu.SMEM),
        out_shape=jax.ShapeDtypeStruct((1, 3), jnp.float32),
        scratch_shapes=[pltpu.VMEM((3, W), jnp.float32)],
        compiler_params=pltpu.CompilerParams(
            dimension_semantics=("arbitrary",),
        ),
    )(y_pred, y_true.astype(jnp.int32))

    scs = jnp.sum(sc_parts.reshape(_NW, 3, 16), axis=(0, 2))

    s = tc_out[0, 0] + scs[0]
    inter = tc_out[0, 1] + scs[1]
    nvalid = tc_out[0, 2] + scs[2]
    union = s + nvalid
    dice = 1.0 - (2.0 * inter + EPS_) / (union + EPS_)
    return dice / C


# speedup vs baseline: 1.5080x; 1.5080x over previous
import jax
import jax.numpy as jnp
from jax import lax
from jax.experimental import pallas as pl
from jax.experimental.pallas import tpu as pltpu

EPS_ = 1.0


def _dice_kernel(x_ref, t_ref, out_ref, acc_ref):
    C, HT, W = x_ref.shape[1:]
    t = t_ref[0]

    def row_group(g, i8):
        tg = t_ref[0, pl.ds(g * 8, 8), :]
        accg = jnp.where(tg == 0, x_ref[0, 0, pl.ds(g * 8, 8), :], 0.0)
        for c in range(1, C):
            accg = accg + jnp.where(tg == c,
                                    x_ref[0, c, pl.ds(g * 8, 8), :], 0.0)
        return i8 + accg

    i8 = lax.fori_loop(0, HT // 8, row_group, jnp.zeros((8, W), jnp.float32))
    i_vec = jnp.sum(i8, axis=0, keepdims=True)

    x2 = x_ref[0].reshape(C * HT, W)
    ones = jnp.ones((1, C * HT), dtype=jnp.float32)
    s_vec = jax.lax.dot_general(
        ones, x2, (((1,), (0,)), ((), ())),
        preferred_element_type=jnp.float32)

    nv_vec = jnp.sum(jnp.where((t >= 0) & (t < C), 1.0, 0.0),
                     axis=0, keepdims=True)

    upd = jnp.concatenate([s_vec, i_vec, nv_vec], axis=0)

    @pl.when(pl.program_id(0) == 0)
    def _init():
        acc_ref[...] = upd

    @pl.when(pl.program_id(0) != 0)
    def _acc():
        acc_ref[...] += upd

    @pl.when(pl.program_id(0) == pl.num_programs(0) - 1)
    def _fin():
        acc = acc_ref[...]
        s = jnp.sum(acc[0])
        inter = jnp.sum(acc[1])
        nvalid = jnp.sum(acc[2])
        union = s + nvalid
        dice = 1.0 - (2.0 * inter + EPS_) / (union + EPS_)
        out_ref[0, 0] = dice / C


def kernel(y_pred, y_true):
    B, C, H, W = y_pred.shape
    HT = 256
    GH = H // HT
    n = B * GH
    out = pl.pallas_call(
        _dice_kernel,
        grid=(n,),
        in_specs=[
            pl.BlockSpec((1, C, HT, W), lambda i: (i // GH, 0, i % GH, 0)),
            pl.BlockSpec((1, HT, W), lambda i: (i // GH, i % GH, 0)),
        ],
        out_specs=pl.BlockSpec((1, 1), lambda i: (0, 0), memory_space=pltpu.SMEM),
        out_shape=jax.ShapeDtypeStruct((1, 1), jnp.float32),
        scratch_shapes=[pltpu.VMEM((3, W), jnp.float32)],
        compiler_params=pltpu.CompilerParams(
            dimension_semantics=("arbitrary",),
        ),
    )(y_pred, y_true.astype(jnp.int32))
    return out[0, 0]
